# baseline (speedup 1.0000x reference)
import jax
import jax.numpy as jnp
from jax.experimental import pallas as pl
from jax.experimental.pallas import tpu as pltpu

B, N, C = 4, 4096, 1024
M = 1024
QB = 512
SCALE = 1.0 / 32.0
LOG2E = 1.4426950408889634
ND = C + 128


def _wkq_body(wk_ref, wq_ref, o_ref):
    o_ref[...] = (jnp.dot(wk_ref[...], wq_ref[...].T,
                          preferred_element_type=jnp.float32)
                  * (SCALE * LOG2E)).astype(jnp.bfloat16)


def _kv_body(x_ref, w_ref, wv_ref, k_ref, v_ref):
    x = x_ref[0]
    k_ref[0] = jnp.dot(x, w_ref[...],
                       preferred_element_type=jnp.float32).astype(jnp.bfloat16)
    v_ref[0, :, :C] = jnp.dot(x, wv_ref[...],
                              preferred_element_type=jnp.float32).astype(jnp.bfloat16)
    v_ref[0, :, C:] = jnp.ones((x.shape[0], 128), jnp.bfloat16)


NCH = 4


def _ne(q, k, v):
    ch = M // NCH
    acc = None
    for i in range(NCH):
        kc = k[i * ch:(i + 1) * ch, :]
        vc = v[i * ch:(i + 1) * ch, :]
        s = jax.lax.dot_general(q, kc, (((1,), (1,)), ((), ())),
                                preferred_element_type=jnp.float32)
        e = jnp.exp2(s).astype(jnp.bfloat16)
        p = jnp.dot(e, vc, preferred_element_type=jnp.float32)
        acc = p if acc is None else acc + p
    return acc


def _fused_body(q0_ref, q1_ref, q2_ref, k0_ref, v0_ref, k1_ref, v1_ref,
                k2_ref, v2_ref, p2_ref, p4_ref, out_ref):
    ne0 = _ne(q0_ref[0], k0_ref[0], v0_ref[0])
    ne1 = _ne(q1_ref[0], k1_ref[0], v1_ref[0])
    ne2 = _ne(q2_ref[0], k2_ref[0], v2_ref[0])
    tot = (ne0
           + jnp.dot(p2_ref[...], ne1.astype(jnp.bfloat16),
                     preferred_element_type=jnp.float32)
           + jnp.dot(p4_ref[...], ne2.astype(jnp.bfloat16),
                     preferred_element_type=jnp.float32))
    out_ref[0] = tot[:, :C] / tot[:, C:C + 1]


def kernel(x, Wq, Wk, Wv):
    wqb = Wq.astype(jnp.bfloat16)
    wkb = Wk.astype(jnp.bfloat16)
    wvb = Wv.astype(jnp.bfloat16)
    xb = x.astype(jnp.bfloat16)

    wkq = pl.pallas_call(
        _wkq_body,
        out_shape=jax.ShapeDtypeStruct((C, C), jnp.bfloat16),
    )(wkb, wqb)

    bn = 512
    k, v = pl.pallas_call(
        _kv_body,
        grid=(B, N // bn),
        in_specs=[
            pl.BlockSpec((1, bn, C), lambda b, i: (b, i, 0)),
            pl.BlockSpec((C, C), lambda b, i: (0, 0)),
            pl.BlockSpec((C, C), lambda b, i: (0, 0)),
        ],
        out_specs=[
            pl.BlockSpec((1, bn, C), lambda b, i: (b, i, 0)),
            pl.BlockSpec((1, bn, ND), lambda b, i: (b, i, 0)),
        ],
        out_shape=[
            jax.ShapeDtypeStruct((B, N, C), jnp.bfloat16),
            jax.ShapeDtypeStruct((B, N, ND), jnp.bfloat16),
        ],
    )(xb, wkq, wvb)

    q = xb
    q1 = q.reshape(B, N // 2, 2 * C)
    q2 = q.reshape(B, N // 4, 4 * C)
    k1 = k.reshape(B, N // 2, 2 * C)
    k2 = k.reshape(B, N // 4, 4 * C)
    v1 = v.reshape(B, N // 2, 2 * ND)
    v2 = v.reshape(B, N // 4, 4 * ND)

    p2 = jnp.eye(QB, dtype=jnp.bfloat16)[:, ::2].copy()
    p4 = jnp.eye(QB, dtype=jnp.bfloat16)[:, ::4].copy()

    tq = M // QB
    out = pl.pallas_call(
        _fused_body,
        grid=(B, 4, tq),
        in_specs=[
            pl.BlockSpec((1, QB, C), lambda b, s, t: (b, s * tq + t, 0)),
            pl.BlockSpec((1, QB // 2, C), lambda b, s, t: (b, s * tq + t, 0)),
            pl.BlockSpec((1, QB // 4, C), lambda b, s, t: (b, s * tq + t, 0)),
            pl.BlockSpec((1, M, C), lambda b, s, t: (b, s, 0)),
            pl.BlockSpec((1, M, ND), lambda b, s, t: (b, s, 0)),
            pl.BlockSpec((1, M, C), lambda b, s, t: (b, s // 2, 0)),
            pl.BlockSpec((1, M, ND), lambda b, s, t: (b, s // 2, 0)),
            pl.BlockSpec((1, M, C), lambda b, s, t: (b, 0, 0)),
            pl.BlockSpec((1, M, ND), lambda b, s, t: (b, 0, 0)),
            pl.BlockSpec((QB, QB // 2), lambda b, s, t: (0, 0)),
            pl.BlockSpec((QB, QB // 4), lambda b, s, t: (0, 0)),
        ],
        out_specs=pl.BlockSpec((1, QB, C), lambda b, s, t: (b, s * tq + t, 0)),
        out_shape=jax.ShapeDtypeStruct((B, N, C), jnp.float32),
        compiler_params=pltpu.CompilerParams(
            vmem_limit_bytes=100 * 1024 * 1024),
    )(q, q1, q2, k, v, k1, v1, k2, v2, p2, p4)
    return out


# speedup vs baseline: 1.1259x; 1.1259x over previous
import jax
import jax.numpy as jnp
from jax.experimental import pallas as pl
from jax.experimental.pallas import tpu as pltpu

B, N, C = 4, 4096, 1024
M = 1024
QB = 512
SCALE = 1.0 / 32.0
LOG2E = 1.4426950408889634
ND = C + 128


def _wkq_body(wk_ref, wq_ref, o_ref):
    o_ref[...] = (jnp.dot(wk_ref[...], wq_ref[...].T,
                          preferred_element_type=jnp.float32)
                  * (SCALE * LOG2E)).astype(jnp.bfloat16)


def _kv_body(x_ref, w_ref, wv_ref, k_ref, v_ref):
    x = x_ref[0]
    k_ref[0] = jnp.dot(x, w_ref[...],
                       preferred_element_type=jnp.float32).astype(jnp.bfloat16)
    v_ref[0] = jnp.dot(x, wv_ref[...],
                       preferred_element_type=jnp.float32).astype(jnp.bfloat16)


NCH = 2


def _ne(q, k, v):
    ch = M // NCH
    acc = None
    accd = None
    for i in range(NCH):
        kc = k[i * ch:(i + 1) * ch, :]
        vc = v[i * ch:(i + 1) * ch, :]
        s = jax.lax.dot_general(q, kc, (((1,), (1,)), ((), ())),
                                preferred_element_type=jnp.float32)
        ef = jnp.exp2(s)
        e = ef.astype(jnp.bfloat16)
        d = jnp.sum(ef, axis=1, keepdims=True)
        p = jnp.dot(e, vc, preferred_element_type=jnp.float32)
        acc = p if acc is None else acc + p
        accd = d if accd is None else accd + d
    return acc, accd


def _up2(a):
    r, w = a.shape
    return jnp.stack([a, jnp.zeros_like(a)], axis=1).reshape(2 * r, w)


def _up4(a):
    r, w = a.shape
    z = jnp.zeros((r, 3, w), dtype=a.dtype)
    return jnp.concatenate([a[:, None, :], z], axis=1).reshape(4 * r, w)


def _fused_body(q0_ref, q1_ref, q2_ref, k0_ref, v0_ref, k1_ref, v1_ref,
                k2_ref, v2_ref, p2_ref, p4_ref, out_ref):
    ne0, d0 = _ne(q0_ref[0], k0_ref[0], v0_ref[0])
    ne1, d1 = _ne(q1_ref[0], k1_ref[0], v1_ref[0])
    ne2, d2 = _ne(q2_ref[0], k2_ref[0], v2_ref[0])
    num = (ne0
           + jnp.dot(p2_ref[...], ne1.astype(jnp.bfloat16),
                     preferred_element_type=jnp.float32)
           + jnp.dot(p4_ref[...], ne2.astype(jnp.bfloat16),
                     preferred_element_type=jnp.float32))
    den = d0 + _up2(d1) + _up4(d2)
    out_ref[0] = num / den


def kernel(x, Wq, Wk, Wv):
    wqb = Wq.astype(jnp.bfloat16)
    wkb = Wk.astype(jnp.bfloat16)
    wvb = Wv.astype(jnp.bfloat16)
    xb = x.astype(jnp.bfloat16)

    wkq = pl.pallas_call(
        _wkq_body,
        out_shape=jax.ShapeDtypeStruct((C, C), jnp.bfloat16),
    )(wkb, wqb)

    bn = 512
    k, v = pl.pallas_call(
        _kv_body,
        grid=(B, N // bn),
        in_specs=[
            pl.BlockSpec((1, bn, C), lambda b, i: (b, i, 0)),
            pl.BlockSpec((C, C), lambda b, i: (0, 0)),
            pl.BlockSpec((C, C), lambda b, i: (0, 0)),
        ],
        out_specs=[
            pl.BlockSpec((1, bn, C), lambda b, i: (b, i, 0)),
            pl.BlockSpec((1, bn, C), lambda b, i: (b, i, 0)),
        ],
        out_shape=[
            jax.ShapeDtypeStruct((B, N, C), jnp.bfloat16),
            jax.ShapeDtypeStruct((B, N, C), jnp.bfloat16),
        ],
    )(xb, wkq, wvb)

    q = xb
    q1 = q.reshape(B, N // 2, 2 * C)
    q2 = q.reshape(B, N // 4, 4 * C)
    k1 = k.reshape(B, N // 2, 2 * C)
    k2 = k.reshape(B, N // 4, 4 * C)
    v1 = v.reshape(B, N // 2, 2 * C)
    v2 = v.reshape(B, N // 4, 4 * C)

    p2 = jnp.eye(QB, dtype=jnp.bfloat16)[:, ::2].copy()
    p4 = jnp.eye(QB, dtype=jnp.bfloat16)[:, ::4].copy()

    tq = M // QB
    out = pl.pallas_call(
        _fused_body,
        grid=(B, 4, tq),
        in_specs=[
            pl.BlockSpec((1, QB, C), lambda b, s, t: (b, s * tq + t, 0)),
            pl.BlockSpec((1, QB // 2, C), lambda b, s, t: (b, s * tq + t, 0)),
            pl.BlockSpec((1, QB // 4, C), lambda b, s, t: (b, s * tq + t, 0)),
            pl.BlockSpec((1, M, C), lambda b, s, t: (b, s, 0)),
            pl.BlockSpec((1, M, C), lambda b, s, t: (b, s, 0)),
            pl.BlockSpec((1, M, C), lambda b, s, t: (b, s // 2, 0)),
            pl.BlockSpec((1, M, C), lambda b, s, t: (b, s // 2, 0)),
            pl.BlockSpec((1, M, C), lambda b, s, t: (b, 0, 0)),
            pl.BlockSpec((1, M, C), lambda b, s, t: (b, 0, 0)),
            pl.BlockSpec((QB, QB // 2), lambda b, s, t: (0, 0)),
            pl.BlockSpec((QB, QB // 4), lambda b, s, t: (0, 0)),
        ],
        out_specs=pl.BlockSpec((1, QB, C), lambda b, s, t: (b, s * tq + t, 0)),
        out_shape=jax.ShapeDtypeStruct((B, N, C), jnp.float32),
        compiler_params=pltpu.CompilerParams(
            vmem_limit_bytes=100 * 1024 * 1024),
    )(q, q1, q2, k, v, k1, v1, k2, v2, p2, p4)
    return out


# speedup vs baseline: 1.1307x; 1.0043x over previous
import jax
import jax.numpy as jnp
from jax.experimental import pallas as pl
from jax.experimental.pallas import tpu as pltpu

B, N, C = 4, 4096, 1024
M = 1024
QB = 512
SCALE = 1.0 / 32.0
LOG2E = 1.4426950408889634
ND = C + 128


def _wkq_body(wk_ref, wq_ref, o_ref):
    o_ref[...] = (jnp.dot(wk_ref[...], wq_ref[...].T,
                          preferred_element_type=jnp.float32)
                  * (SCALE * LOG2E)).astype(jnp.bfloat16)


def _kv_body(x_ref, w_ref, wv_ref, k_ref, v_ref):
    x = x_ref[0]
    k_ref[0] = jnp.dot(x, w_ref[...],
                       preferred_element_type=jnp.float32).astype(jnp.bfloat16)
    v_ref[0] = jnp.dot(x, wv_ref[...],
                       preferred_element_type=jnp.float32).astype(jnp.bfloat16)


NCH = 2


def _ne(q, k, v):
    ch = M // NCH
    acc = None
    accd = None
    for i in range(NCH):
        kc = k[i * ch:(i + 1) * ch, :]
        vc = v[i * ch:(i + 1) * ch, :]
        s = jax.lax.dot_general(q, kc, (((1,), (1,)), ((), ())),
                                preferred_element_type=jnp.float32)
        ef = jnp.exp2(s)
        e = ef.astype(jnp.bfloat16)
        d = jnp.sum(ef, axis=1, keepdims=True)
        p = jnp.dot(e, vc, preferred_element_type=jnp.float32)
        acc = p if acc is None else acc + p
        accd = d if accd is None else accd + d
    return acc, accd


def _up2(a):
    r, w = a.shape
    return jnp.stack([a, jnp.zeros_like(a)], axis=1).reshape(2 * r, w)


def _up4(a):
    r, w = a.shape
    z = jnp.zeros((r, 3, w), dtype=a.dtype)
    return jnp.concatenate([a[:, None, :], z], axis=1).reshape(4 * r, w)


def _fused_body(q0_ref, q1_ref, q2_ref, k0_ref, v0_ref, k1_ref, v1_ref,
                k2_ref, v2_ref, p2_ref, p4_ref, out_ref):
    ne0, d0 = _ne(q0_ref[0], k0_ref[0], v0_ref[0])
    ne1, d1 = _ne(q1_ref[0], k1_ref[0], v1_ref[0])
    ne2, d2 = _ne(q2_ref[0], k2_ref[0], v2_ref[0])
    num = (ne0
           + jnp.dot(p2_ref[...], ne1.astype(jnp.bfloat16),
                     preferred_element_type=jnp.float32)
           + jnp.dot(p4_ref[...], ne2.astype(jnp.bfloat16),
                     preferred_element_type=jnp.float32))
    den = d0 + _up2(d1) + _up4(d2)
    out_ref[0] = num / den


def kernel(x, Wq, Wk, Wv):
    wqb = Wq.astype(jnp.bfloat16)
    wkb = Wk.astype(jnp.bfloat16)
    wvb = Wv.astype(jnp.bfloat16)
    xb = x.astype(jnp.bfloat16)

    wkq = pl.pallas_call(
        _wkq_body,
        out_shape=jax.ShapeDtypeStruct((C, C), jnp.bfloat16),
    )(wkb, wqb)

    bn = 1024
    k, v = pl.pallas_call(
        _kv_body,
        grid=(B, N // bn),
        in_specs=[
            pl.BlockSpec((1, bn, C), lambda b, i: (b, i, 0)),
            pl.BlockSpec((C, C), lambda b, i: (0, 0)),
            pl.BlockSpec((C, C), lambda b, i: (0, 0)),
        ],
        out_specs=[
            pl.BlockSpec((1, bn, C), lambda b, i: (b, i, 0)),
            pl.BlockSpec((1, bn, C), lambda b, i: (b, i, 0)),
        ],
        out_shape=[
            jax.ShapeDtypeStruct((B, N, C), jnp.bfloat16),
            jax.ShapeDtypeStruct((B, N, C), jnp.bfloat16),
        ],
    )(xb, wkq, wvb)

    q = xb
    q1 = q.reshape(B, N // 2, 2 * C)
    q2 = q.reshape(B, N // 4, 4 * C)
    k1 = k.reshape(B, N // 2, 2 * C)
    k2 = k.reshape(B, N // 4, 4 * C)
    v1 = v.reshape(B, N // 2, 2 * C)
    v2 = v.reshape(B, N // 4, 4 * C)

    p2 = jnp.eye(QB, dtype=jnp.bfloat16)[:, ::2].copy()
    p4 = jnp.eye(QB, dtype=jnp.bfloat16)[:, ::4].copy()

    tq = M // QB
    out = pl.pallas_call(
        _fused_body,
        grid=(B, 4, tq),
        in_specs=[
            pl.BlockSpec((1, QB, C), lambda b, s, t: (b, s * tq + t, 0)),
            pl.BlockSpec((1, QB // 2, C), lambda b, s, t: (b, s * tq + t, 0)),
            pl.BlockSpec((1, QB // 4, C), lambda b, s, t: (b, s * tq + t, 0)),
            pl.BlockSpec((1, M, C), lambda b, s, t: (b, s, 0)),
            pl.BlockSpec((1, M, C), lambda b, s, t: (b, s, 0)),
            pl.BlockSpec((1, M, C), lambda b, s, t: (b, s // 2, 0)),
            pl.BlockSpec((1, M, C), lambda b, s, t: (b, s // 2, 0)),
            pl.BlockSpec((1, M, C), lambda b, s, t: (b, 0, 0)),
            pl.BlockSpec((1, M, C), lambda b, s, t: (b, 0, 0)),
            pl.BlockSpec((QB, QB // 2), lambda b, s, t: (0, 0)),
            pl.BlockSpec((QB, QB // 4), lambda b, s, t: (0, 0)),
        ],
        out_specs=pl.BlockSpec((1, QB, C), lambda b, s, t: (b, s * tq + t, 0)),
        out_shape=jax.ShapeDtypeStruct((B, N, C), jnp.float32),
        compiler_params=pltpu.CompilerParams(
            vmem_limit_bytes=100 * 1024 * 1024),
    )(q, q1, q2, k, v, k1, v1, k2, v2, p2, p4)
    return out


# speedup vs baseline: 1.1369x; 1.0055x over previous
import jax
import jax.numpy as jnp
from jax.experimental import pallas as pl
from jax.experimental.pallas import tpu as pltpu

B, N, C = 4, 4096, 1024
M = 1024
QB = 512
SCALE = 1.0 / 32.0
LOG2E = 1.4426950408889634
ND = C + 128


def _wkq_body(wk_ref, wq_ref, o_ref):
    o_ref[...] = (jnp.dot(wk_ref[...], wq_ref[...].T,
                          preferred_element_type=jnp.float32)
                  * (SCALE * LOG2E)).astype(jnp.bfloat16)


def _kv_body(x_ref, w_ref, wv_ref, k_ref, v_ref):
    x = x_ref[0]
    k_ref[0] = jnp.dot(x, w_ref[...],
                       preferred_element_type=jnp.float32).astype(jnp.bfloat16)
    v_ref[0] = jnp.dot(x, wv_ref[...],
                       preferred_element_type=jnp.float32).astype(jnp.bfloat16)


NCH = 2


def _ne(q, k, v, nch=NCH):
    ch = M // nch
    acc = None
    accd = None
    for i in range(nch):
        kc = k[i * ch:(i + 1) * ch, :]
        vc = v[i * ch:(i + 1) * ch, :]
        s = jax.lax.dot_general(q, kc, (((1,), (1,)), ((), ())),
                                preferred_element_type=jnp.float32)
        ef = jnp.exp2(s)
        e = ef.astype(jnp.bfloat16)
        d = jnp.sum(ef, axis=1, keepdims=True)
        p = jnp.dot(e, vc, preferred_element_type=jnp.float32)
        acc = p if acc is None else acc + p
        accd = d if accd is None else accd + d
    return acc, accd


def _up2(a):
    r, w = a.shape
    return jnp.stack([a, jnp.zeros_like(a)], axis=1).reshape(2 * r, w)


def _up4(a):
    r, w = a.shape
    z = jnp.zeros((r, 3, w), dtype=a.dtype)
    return jnp.concatenate([a[:, None, :], z], axis=1).reshape(4 * r, w)


def _fused_body(q0_ref, q1_ref, q2_ref, k0_ref, v0_ref, k1_ref, v1_ref,
                k2_ref, v2_ref, p2_ref, p4_ref, out_ref):
    ne0, d0 = _ne(q0_ref[0], k0_ref[0], v0_ref[0])
    ne1, d1 = _ne(q1_ref[0], k1_ref[0], v1_ref[0], 1)
    ne2, d2 = _ne(q2_ref[0], k2_ref[0], v2_ref[0], 1)
    num = (ne0
           + jnp.dot(p2_ref[...], ne1.astype(jnp.bfloat16),
                     preferred_element_type=jnp.float32)
           + jnp.dot(p4_ref[...], ne2.astype(jnp.bfloat16),
                     preferred_element_type=jnp.float32))
    den = d0 + _up2(d1) + _up4(d2)
    out_ref[0] = num / den


def kernel(x, Wq, Wk, Wv):
    wqb = Wq.astype(jnp.bfloat16)
    wkb = Wk.astype(jnp.bfloat16)
    wvb = Wv.astype(jnp.bfloat16)
    xb = x.astype(jnp.bfloat16)

    wkq = pl.pallas_call(
        _wkq_body,
        out_shape=jax.ShapeDtypeStruct((C, C), jnp.bfloat16),
    )(wkb, wqb)

    bn = 1024
    k, v = pl.pallas_call(
        _kv_body,
        grid=(B, N // bn),
        in_specs=[
            pl.BlockSpec((1, bn, C), lambda b, i: (b, i, 0)),
            pl.BlockSpec((C, C), lambda b, i: (0, 0)),
            pl.BlockSpec((C, C), lambda b, i: (0, 0)),
        ],
        out_specs=[
            pl.BlockSpec((1, bn, C), lambda b, i: (b, i, 0)),
            pl.BlockSpec((1, bn, C), lambda b, i: (b, i, 0)),
        ],
        out_shape=[
            jax.ShapeDtypeStruct((B, N, C), jnp.bfloat16),
            jax.ShapeDtypeStruct((B, N, C), jnp.bfloat16),
        ],
    )(xb, wkq, wvb)

    q = xb
    q1 = q.reshape(B, N // 2, 2 * C)
    q2 = q.reshape(B, N // 4, 4 * C)
    k1 = k.reshape(B, N // 2, 2 * C)
    k2 = k.reshape(B, N // 4, 4 * C)
    v1 = v.reshape(B, N // 2, 2 * C)
    v2 = v.reshape(B, N // 4, 4 * C)

    p2 = jnp.eye(QB, dtype=jnp.bfloat16)[:, ::2].copy()
    p4 = jnp.eye(QB, dtype=jnp.bfloat16)[:, ::4].copy()

    tq = M // QB
    out = pl.pallas_call(
        _fused_body,
        grid=(B, 4, tq),
        in_specs=[
            pl.BlockSpec((1, QB, C), lambda b, s, t: (b, s * tq + t, 0)),
            pl.BlockSpec((1, QB // 2, C), lambda b, s, t: (b, s * tq + t, 0)),
            pl.BlockSpec((1, QB // 4, C), lambda b, s, t: (b, s * tq + t, 0)),
            pl.BlockSpec((1, M, C), lambda b, s, t: (b, s, 0)),
            pl.BlockSpec((1, M, C), lambda b, s, t: (b, s, 0)),
            pl.BlockSpec((1, M, C), lambda b, s, t: (b, s // 2, 0)),
            pl.BlockSpec((1, M, C), lambda b, s, t: (b, s // 2, 0)),
            pl.BlockSpec((1, M, C), lambda b, s, t: (b, 0, 0)),
            pl.BlockSpec((1, M, C), lambda b, s, t: (b, 0, 0)),
            pl.BlockSpec((QB, QB // 2), lambda b, s, t: (0, 0)),
            pl.BlockSpec((QB, QB // 4), lambda b, s, t: (0, 0)),
        ],
        out_specs=pl.BlockSpec((1, QB, C), lambda b, s, t: (b, s * tq + t, 0)),
        out_shape=jax.ShapeDtypeStruct((B, N, C), jnp.float32),
        compiler_params=pltpu.CompilerParams(
            vmem_limit_bytes=100 * 1024 * 1024),
    )(q, q1, q2, k, v, k1, v1, k2, v2, p2, p4)
    return out


# speedup vs baseline: 1.1374x; 1.0004x over previous
import jax
import jax.numpy as jnp
from jax.experimental import pallas as pl
from jax.experimental.pallas import tpu as pltpu

B, N, C = 4, 4096, 1024
M = 1024
QB = 512
SCALE = 1.0 / 32.0
LOG2E = 1.4426950408889634


def _wkq_body(wk_ref, wq_ref, o_ref):
    o_ref[...] = (jnp.dot(wk_ref[...], wq_ref[...].T,
                          preferred_element_type=jnp.float32)
                  * (SCALE * LOG2E)).astype(jnp.bfloat16)


def _kv_body(x_ref, w_ref, wv_ref, k_ref, v_ref):
    x = x_ref[0]
    k_ref[0] = jnp.dot(x, w_ref[...],
                       preferred_element_type=jnp.float32).astype(jnp.bfloat16)
    v_ref[0] = jnp.dot(x, wv_ref[...],
                       preferred_element_type=jnp.float32).astype(jnp.bfloat16)


NCH = 2


def _ne(q, k, v, nch=NCH):
    ch = M // nch
    acc = None
    accd = None
    for i in range(nch):
        kc = k[i * ch:(i + 1) * ch, :]
        vc = v[i * ch:(i + 1) * ch, :]
        s = jax.lax.dot_general(q, kc, (((1,), (1,)), ((), ())),
                                preferred_element_type=jnp.float32)
        ef = jnp.exp2(s)
        e = ef.astype(jnp.bfloat16)
        d = jnp.sum(ef, axis=1, keepdims=True)
        p = jnp.dot(e, vc, preferred_element_type=jnp.float32)
        acc = p if acc is None else acc + p
        accd = d if accd is None else accd + d
    return acc, accd


def _up2(a):
    r, w = a.shape
    return jnp.stack([a, jnp.zeros_like(a)], axis=1).reshape(2 * r, w)


def _up4(a):
    r, w = a.shape
    z = jnp.zeros((r, 3, w), dtype=a.dtype)
    return jnp.concatenate([a[:, None, :], z], axis=1).reshape(4 * r, w)


def _fused_body(q0_ref, q1_ref, q2_ref, k0_ref, v0_ref, k1_ref, v1_ref,
                k2_ref, v2_ref, p2_ref, p4_ref, out_ref):
    ne0, d0 = _ne(q0_ref[0], k0_ref[0], v0_ref[0])
    ne1, d1 = _ne(q1_ref[0], k1_ref[0], v1_ref[0], 1)
    ne2, d2 = _ne(q2_ref[0], k2_ref[0], v2_ref[0], 1)
    num = (ne0
           + jnp.dot(p2_ref[...], ne1.astype(jnp.bfloat16),
                     preferred_element_type=jnp.float32)
           + jnp.dot(p4_ref[...], ne2.astype(jnp.bfloat16),
                     preferred_element_type=jnp.float32))
    den = d0 + _up2(d1) + _up4(d2)
    out_ref[0] = num / den


def kernel(x, Wq, Wk, Wv):
    wqb = Wq.astype(jnp.bfloat16)
    wkb = Wk.astype(jnp.bfloat16)
    wvb = Wv.astype(jnp.bfloat16)
    xb = x.astype(jnp.bfloat16)

    wkq = pl.pallas_call(
        _wkq_body,
        out_shape=jax.ShapeDtypeStruct((C, C), jnp.bfloat16),
    )(wkb, wqb)

    bn = 1024
    k, v = pl.pallas_call(
        _kv_body,
        grid=(B, N // bn),
        in_specs=[
            pl.BlockSpec((1, bn, C), lambda b, i: (b, i, 0)),
            pl.BlockSpec((C, C), lambda b, i: (0, 0)),
            pl.BlockSpec((C, C), lambda b, i: (0, 0)),
        ],
        out_specs=[
            pl.BlockSpec((1, bn, C), lambda b, i: (b, i, 0)),
            pl.BlockSpec((1, bn, C), lambda b, i: (b, i, 0)),
        ],
        out_shape=[
            jax.ShapeDtypeStruct((B, N, C), jnp.bfloat16),
            jax.ShapeDtypeStruct((B, N, C), jnp.bfloat16),
        ],
    )(xb, wkq, wvb)

    q = xb
    q1 = q.reshape(B, N // 2, 2 * C)
    q2 = q.reshape(B, N // 4, 4 * C)
    k1 = k.reshape(B, N // 2, 2 * C)
    k2 = k.reshape(B, N // 4, 4 * C)
    v1 = v.reshape(B, N // 2, 2 * C)
    v2 = v.reshape(B, N // 4, 4 * C)

    p2 = jnp.eye(QB, dtype=jnp.bfloat16)[:, ::2].copy()
    p4 = jnp.eye(QB, dtype=jnp.bfloat16)[:, ::4].copy()

    tq = M // QB
    out = pl.pallas_call(
        _fused_body,
        grid=(B, 4, tq),
        in_specs=[
            pl.BlockSpec((1, QB, C), lambda b, s, t: (b, s * tq + t, 0)),
            pl.BlockSpec((1, QB // 2, C), lambda b, s, t: (b, s * tq + t, 0)),
            pl.BlockSpec((1, QB // 4, C), lambda b, s, t: (b, s * tq + t, 0)),
            pl.BlockSpec((1, M, C), lambda b, s, t: (b, s, 0)),
            pl.BlockSpec((1, M, C), lambda b, s, t: (b, s, 0)),
            pl.BlockSpec((1, M, C), lambda b, s, t: (b, s // 2, 0)),
            pl.BlockSpec((1, M, C), lambda b, s, t: (b, s // 2, 0)),
            pl.BlockSpec((1, M, C), lambda b, s, t: (b, 0, 0)),
            pl.BlockSpec((1, M, C), lambda b, s, t: (b, 0, 0)),
            pl.BlockSpec((QB, QB // 2), lambda b, s, t: (0, 0)),
            pl.BlockSpec((QB, QB // 4), lambda b, s, t: (0, 0)),
        ],
        out_specs=pl.BlockSpec((1, QB, C), lambda b, s, t: (b, s * tq + t, 0)),
        out_shape=jax.ShapeDtypeStruct((B, N, C), jnp.float32),
        compiler_params=pltpu.CompilerParams(
            vmem_limit_bytes=100 * 1024 * 1024),
    )(q, q1, q2, k, v, k1, v1, k2, v2, p2, p4)
    return out
